# baseline (speedup 1.0000x reference)
import functools

import jax
import jax.numpy as jnp
from jax import lax
from jax.experimental import pallas as pl
from jax.experimental.pallas import tpu as pltpu
from jax.experimental.pallas import tpu_sc as plsc

_B = 16384
_D = 16
_NC = 2
_NS = 16
_NW = _NC * _NS
_CHUNK = 128
_ROWS_PER_W = _B // _NW
_NCH = _ROWS_PER_W // _CHUNK


def _sc_gather_body(uidx_hbm, iidx_hbm, w_hbm, h_hbm, h1_hbm,
                    u_out, v_out, v1_out,
                    uidx_v, iidx_v, u_rows, v_rows, v1_rows, sem):
    wid = lax.axis_index("s") * _NC + lax.axis_index("c")
    base = wid * _ROWS_PER_W
    pltpu.sync_copy(uidx_hbm.at[wid], uidx_v)
    pltpu.sync_copy(iidx_hbm.at[wid], iidx_v)
    copies = []
    for j in range(_NCH):
        dst = pl.ds(j * _CHUNK, _CHUNK)
        copies.append(pltpu.async_copy(w_hbm.at[uidx_v.at[j]], u_rows.at[dst], sem))
        copies.append(pltpu.async_copy(h_hbm.at[iidx_v.at[j]], v_rows.at[dst], sem))
        copies.append(pltpu.async_copy(h1_hbm.at[iidx_v.at[j]], v1_rows.at[dst], sem))
    for c in copies:
        c.wait()
    out_sl = pl.ds(base, _ROWS_PER_W)
    pltpu.sync_copy(u_rows, u_out.at[out_sl])
    pltpu.sync_copy(v_rows, v_out.at[out_sl])
    pltpu.sync_copy(v1_rows, v1_out.at[out_sl])


@functools.lru_cache(maxsize=None)
def _sc_gather():
    return functools.partial(
        pl.kernel,
        out_type=[jax.ShapeDtypeStruct((_B, _D), jnp.float32)] * 3,
        mesh=plsc.VectorSubcoreMesh(core_axis_name="c", subcore_axis_name="s"),
        compiler_params=pltpu.CompilerParams(use_tc_tiling_on_sc=False),
        scratch_types=[
            pltpu.VMEM((_NCH, _CHUNK), jnp.int32),
            pltpu.VMEM((_NCH, _CHUNK), jnp.int32),
            pltpu.VMEM((_ROWS_PER_W, _D), jnp.float32),
            pltpu.VMEM((_ROWS_PER_W, _D), jnp.float32),
            pltpu.VMEM((_ROWS_PER_W, _D), jnp.float32),
            pltpu.SemaphoreType.DMA,
        ],
    )(_sc_gather_body)


def _mlp_body(drug_ref, u_ref, v_ref, v1_ref, ww1_ref, wb1_ref, ww2_ref,
              wb2_ref, dw1_ref, db1_ref, dw2_ref, g_ref, out_ref):
    drug = drug_ref[...]
    wh = jnp.maximum(
        jnp.dot(drug, ww1_ref[...], preferred_element_type=jnp.float32)
        + wb1_ref[...], 0.0)
    wide = (jnp.dot(wh, ww2_ref[...], preferred_element_type=jnp.float32)
            + wb2_ref[...]) * v1_ref[...]
    wide_t = jnp.sum(wide, axis=1, keepdims=True)
    z = jnp.concatenate([u_ref[...], v_ref[...]], axis=1)
    h = jax.nn.sigmoid(
        jnp.dot(z, dw1_ref[...], preferred_element_type=jnp.float32)
        + db1_ref[...])
    dnn = jnp.dot(h, dw2_ref[...], preferred_element_type=jnp.float32)
    gw = g_ref[0, 0]
    gb = g_ref[0, 1]
    out_ref[...] = jax.nn.sigmoid(wide_t * gw + gb + dnn)[:, 0]


def _mlp_call(blk):
    grid = _B // blk
    full = lambda shape: pl.BlockSpec(shape, lambda i: (0, 0))
    return pl.pallas_call(
        _mlp_body,
        grid=(grid,),
        in_specs=[
            pl.BlockSpec((blk, 256), lambda i: (i, 0)),
            pl.BlockSpec((blk, _D), lambda i: (i, 0)),
            pl.BlockSpec((blk, _D), lambda i: (i, 0)),
            pl.BlockSpec((blk, _D), lambda i: (i, 0)),
            full((256, 64)),
            full((1, 64)),
            full((64, _D)),
            full((1, _D)),
            full((2 * _D, _D)),
            full((1, _D)),
            full((_D, 1)),
            full((1, 2)),
        ],
        out_specs=pl.BlockSpec((blk,), lambda i: (i,)),
        out_shape=jax.ShapeDtypeStruct((_B,), jnp.float32),
    )


def kernel(x, drug_features_x, W, H, H1, wide_w1, wide_b1, wide_w2, wide_b2,
           deep_w1, deep_b1, deep_w2, g_w, g_b):
    xi = x.astype(jnp.int32)
    uidx = xi[:, 0].reshape(_NW, _NCH, _CHUNK)
    iidx = xi[:, 1].reshape(_NW, _NCH, _CHUNK)
    u_emb, v_emb, v1_emb = _sc_gather()(uidx, iidx, W, H, H1)
    g = jnp.concatenate([g_w.reshape(1, 1), g_b.reshape(1, 1)], axis=1)
    out = _mlp_call(2048)(
        drug_features_x, u_emb, v_emb, v1_emb,
        wide_w1, wide_b1.reshape(1, 64), wide_w2, wide_b2.reshape(1, _D),
        deep_w1, deep_b1.reshape(1, _D), deep_w2, g)
    return out
